# baseline (speedup 1.0000x reference)
import jax
import jax.numpy as jnp
from jax import lax
from jax.experimental import pallas as pl
from jax.experimental.pallas import tpu as pltpu
from jax.experimental.pallas import tpu_sc as plsc

NUM_CLASSES = 256
GRID = 64
N = GRID * GRID * GRID
NUM_CORES = 2
NUM_SUBCORES = 16
NUM_WORKERS = NUM_CORES * NUM_SUBCORES
PER_WORKER = N // NUM_WORKERS
CHUNK = 256
STEPS = PER_WORKER // CHUNK
LANES = 16


def _onehot_body(vox_hbm, out_hbm, vox_v, tile_v):
    cid = lax.axis_index("c")
    sid = lax.axis_index("s")
    wid = sid * NUM_CORES + cid
    base = wid * PER_WORKER

    zeros16 = jnp.zeros((LANES,), jnp.float32)
    ones16 = jnp.full((LANES,), 1.0, jnp.float32)
    iota16 = lax.iota(jnp.int32, LANES)

    def _zero_row(r, _):
        for k in range(CHUNK // LANES):
            tile_v[r, pl.ds(k * LANES, LANES)] = zeros16
        return 0

    lax.fori_loop(0, NUM_CLASSES, _zero_row, 0)

    def _step(j, _):
        off = pl.multiple_of(base + j * CHUNK, CHUNK)
        pltpu.sync_copy(vox_hbm.at[pl.ds(off, CHUNK)], vox_v)
        for k in range(CHUNK // LANES):
            rows = vox_v[pl.ds(k * LANES, LANES)]
            cols = iota16 + (k * LANES)
            plsc.store_scatter(tile_v, [rows, cols], ones16)
        pltpu.sync_copy(tile_v, out_hbm.at[:, pl.ds(off, CHUNK)])
        for k in range(CHUNK // LANES):
            rows = vox_v[pl.ds(k * LANES, LANES)]
            cols = iota16 + (k * LANES)
            plsc.store_scatter(tile_v, [rows, cols], zeros16)
        return 0

    lax.fori_loop(0, STEPS, _step, 0)


def kernel(voxels):
    vox = voxels.reshape(N).astype(jnp.int32)
    mesh = plsc.VectorSubcoreMesh(
        core_axis_name="c",
        subcore_axis_name="s",
        num_cores=NUM_CORES,
        num_subcores=NUM_SUBCORES,
    )
    out = pl.kernel(
        _onehot_body,
        out_type=jax.ShapeDtypeStruct((NUM_CLASSES, N), jnp.float32),
        mesh=mesh,
        scratch_types=[
            pltpu.VMEM((CHUNK,), jnp.int32),
            pltpu.VMEM((NUM_CLASSES, CHUNK), jnp.float32),
        ],
        compiler_params=pltpu.CompilerParams(
            use_tc_tiling_on_sc=False, needs_layout_passes=False
        ),
    )(vox)
    return out.reshape(NUM_CLASSES, GRID, GRID, GRID)
